# baseline (speedup 1.0000x reference)
import functools

import jax
import jax.numpy as jnp
from jax import lax
from jax.experimental import pallas as pl
from jax.experimental.pallas import tpu as pltpu
from jax.experimental.pallas import tpu_sc as plsc

DIM = 64
MARGIN = 6.0
NC, NS = 2, 16
NW = NC * NS
IW = 128


def _sc_gather(heads, rels, tails, ent_re, ent_im, rel_ph, total):
    ch = heads.shape[1]
    b_per_w = ch * IW
    mesh = plsc.VectorSubcoreMesh(core_axis_name="c", subcore_axis_name="s")
    row_t = jax.ShapeDtypeStruct((total, DIM), jnp.float32)

    @functools.partial(
        pl.kernel, mesh=mesh,
        out_type=[row_t] * 5,
        compiler_params=pltpu.CompilerParams(use_tc_tiling_on_sc=False),
        scratch_types=[pltpu.VMEM((ch, IW), jnp.int32)] * 3
        + [pltpu.VMEM((b_per_w, DIM), jnp.float32),
           pltpu.SemaphoreType.DMA],
    )
    def k(h_hbm, r_hbm, t_hbm, ere_hbm, eim_hbm, ph_hbm,
          o_hre, o_him, o_tre, o_tim, o_ph,
          hidx, ridx, tidx, rows, sem):
        wid = lax.axis_index("s") * NC + lax.axis_index("c")
        base = wid * b_per_w
        pltpu.sync_copy(h_hbm.at[wid], hidx)
        pltpu.sync_copy(r_hbm.at[wid], ridx)
        pltpu.sync_copy(t_hbm.at[wid], tidx)

        def gather_one(idx_v, table_hbm, out_hbm):
            copies = []
            for j in range(ch):
                copies.append(pltpu.async_copy(
                    table_hbm.at[idx_v.at[j]],
                    rows.at[pl.ds(j * IW, IW)], sem))
            for c in copies:
                c.wait()
            pltpu.sync_copy(rows, out_hbm.at[pl.ds(base, b_per_w)])

        gather_one(hidx, ere_hbm, o_hre)
        gather_one(hidx, eim_hbm, o_him)
        gather_one(tidx, ere_hbm, o_tre)
        gather_one(tidx, eim_hbm, o_tim)
        gather_one(ridx, ph_hbm, o_ph)

    return k(heads, rels, tails, ent_re, ent_im, rel_ph)


def _tc_loss(hre, him, tre, tim, ph, batch):
    w = 1024
    g = batch // w

    def body(hre_p, him_p, tre_p, tim_p, ph_p,
             hre_n, him_n, tre_n, tim_n, ph_n, out):
        def mag(a, b, c, d, p):
            rre = jnp.cos(p)
            rim = jnp.sin(p)
            dre = a * rre - b * rim - c
            dim = a * rim + b * rre - d
            return jnp.sum(jnp.sqrt(dre * dre + dim * dim + 1e-9), axis=-1)

        ms = jnp.maximum(MARGIN - mag(hre_n[...], him_n[...], tre_n[...],
                                      tim_n[...], ph_n[...])
                         + mag(hre_p[...], him_p[...], tre_p[...],
                               tim_p[...], ph_p[...]), 0.0)
        i = pl.program_id(0)

        @pl.when(i == 0)
        def _():
            out[...] = jnp.zeros((1, 1), jnp.float32)

        out[...] += jnp.sum(ms).reshape(1, 1)

        @pl.when(i == g - 1)
        def _():
            out[...] = out[...] / batch

    pos_spec = pl.BlockSpec((w, DIM), lambda i: (i, 0))
    neg_spec = pl.BlockSpec((w, DIM), lambda i: (i + g, 0))
    out = pl.pallas_call(
        body,
        grid=(g,),
        in_specs=[pos_spec] * 5 + [neg_spec] * 5,
        out_specs=pl.BlockSpec((1, 1), lambda i: (0, 0)),
        out_shape=jax.ShapeDtypeStruct((1, 1), jnp.float32),
    )(hre, him, tre, tim, ph, hre, him, tre, tim, ph)
    return out[0, 0]


def kernel(positive_triples, negative_triples, entity_re, entity_im,
           relation_phase):
    batch = positive_triples.shape[0]
    total = 2 * batch
    ch = total // (NW * IW)
    pt = positive_triples.astype(jnp.int32)
    nt = negative_triples.astype(jnp.int32)
    heads = jnp.concatenate([pt[:, 0], nt[:, 0]]).reshape(NW, ch, IW)
    rels = jnp.concatenate([pt[:, 1], nt[:, 1]]).reshape(NW, ch, IW)
    tails = jnp.concatenate([pt[:, 2], nt[:, 2]]).reshape(NW, ch, IW)
    hre, him, tre, tim, ph = _sc_gather(heads, rels, tails, entity_re,
                                        entity_im, relation_phase, total)
    return _tc_loss(hre, him, tre, tim, ph, batch)
